# baseline (speedup 1.0000x reference)
import jax
import jax.numpy as jnp
from jax.experimental import pallas as pl
from jax.experimental.pallas import tpu as pltpu

NUM_CLASSES = 21
OVERLAP_THRESH = 0.5
NEG_POS_RATIO = 3
ARM_VARIANCE = (0.1, 0.2)
VARIANCE = (0.1, 0.2)
POS_PRIOR_THRESHOLD = 0.01
T = 8
P = 16320
L = P // 8


def _channels(block, c):
    x3 = block.T.reshape(8, c, L)
    return [x3[:, ch, :] for ch in range(c)]


def _sortable_key(v):
    b = jax.lax.bitcast_convert_type(v, jnp.int32)
    return jnp.where(b >= 0, b, b ^ jnp.int32(0x7FFFFFFF))


def _odm_kernel(tgt_ref, bi_loc_ref, bi_conf_ref, multi_loc_ref,
                multi_conf_ref, priors_ref, ll_ref, lc_ref, np_ref):
    f32 = jnp.float32

    @pl.when(pl.program_id(0) == 0)
    def _init():
        ll_ref[0, 0] = f32(0.0)
        lc_ref[0, 0] = f32(0.0)
        np_ref[0, 0] = f32(0.0)

    al = _channels(bi_loc_ref[0], 4)
    ml = _channels(multi_loc_ref[0], 4)
    pr = [priors_ref[ch] for ch in range(4)]

    idx = (jax.lax.broadcasted_iota(jnp.int32, (8, L), 1) * 8
           + jax.lax.broadcasted_iota(jnp.int32, (8, L), 0))

    cx = pr[0] + al[0] * (ARM_VARIANCE[0] * pr[2])
    cy = pr[1] + al[1] * (ARM_VARIANCE[0] * pr[3])
    w = pr[2] * jnp.exp(al[2] * ARM_VARIANCE[1])
    h = pr[3] * jnp.exp(al[3] * ARM_VARIANCE[1])
    x0 = cx - w / 2.0
    y0 = cy - h / 2.0
    x1 = cx + w / 2.0
    y1 = cy + h / 2.0
    area_p = (x1 - x0) * (y1 - y0)

    tx0 = [tgt_ref[0, t, 0] for t in range(T)]
    ty0 = [tgt_ref[0, t, 1] for t in range(T)]
    tx1 = [tgt_ref[0, t, 2] for t in range(T)]
    ty1 = [tgt_ref[0, t, 3] for t in range(T)]
    tlb = [tgt_ref[0, t, 4] for t in range(T)]

    ious = []
    for t in range(T):
        iw = jnp.maximum(jnp.minimum(tx1[t], x1) - jnp.maximum(tx0[t], x0), 0.0)
        ih = jnp.maximum(jnp.minimum(ty1[t], y1) - jnp.maximum(ty0[t], y0), 0.0)
        inter = iw * ih
        area_t = (tx1[t] - tx0[t]) * (ty1[t] - ty0[t])
        ious.append(inter / (area_t + area_p - inter))

    bt_over = ious[0]
    bt_idx = jnp.zeros((8, L), jnp.int32)
    for t in range(1, T):
        upd = ious[t] > bt_over
        bt_idx = jnp.where(upd, jnp.int32(t), bt_idx)
        bt_over = jnp.where(upd, ious[t], bt_over)

    for t in range(T):
        mval = jnp.max(ious[t])
        bpi = jnp.min(jnp.where(ious[t] == mval, idx, jnp.int32(P)))
        hit = idx == bpi
        bt_over = jnp.where(hit, f32(2.0), bt_over)
        bt_idx = jnp.where(hit, jnp.int32(t), bt_idx)

    m0 = jnp.zeros((8, L), f32)
    m1 = jnp.zeros((8, L), f32)
    m2 = jnp.zeros((8, L), f32)
    m3 = jnp.zeros((8, L), f32)
    lbl = jnp.zeros((8, L), f32)
    for t in range(T):
        sel_t = bt_idx == t
        m0 = jnp.where(sel_t, tx0[t], m0)
        m1 = jnp.where(sel_t, ty0[t], m1)
        m2 = jnp.where(sel_t, tx1[t], m2)
        m3 = jnp.where(sel_t, ty1[t], m3)
        lbl = jnp.where(sel_t, tlb[t], lbl)

    conf_t = lbl.astype(jnp.int32) + 1
    conf_t = jnp.where(bt_over < OVERLAP_THRESH, 0, conf_t)
    pos = conf_t > 0
    posf = pos.astype(f32)
    num_pos = jnp.sum(conf_t > 0, dtype=jnp.int32)

    g0 = ((m0 + m2) / 2.0 - cx) / (VARIANCE[0] * w)
    g1 = ((m1 + m3) / 2.0 - cy) / (VARIANCE[0] * h)
    g2 = jnp.log((m2 - m0) / w) / VARIANCE[1]
    g3 = jnp.log((m3 - m1) / h) / VARIANCE[1]

    loss_l = f32(0.0)
    for d, g in ((ml[0], g0), (ml[1], g1), (ml[2], g2), (ml[3], g3)):
        df = d - g
        ad = jnp.abs(df)
        sl1 = jnp.where(ad < 1.0, 0.5 * df * df, ad - 0.5)
        loss_l = loss_l + jnp.sum(sl1 * posf)

    mc3 = multi_conf_ref[0].T.reshape(8, NUM_CLASSES, L)
    cmax = jnp.max(mc3, axis=1)
    ez = jnp.exp(mc3 - cmax[:, None, :])
    sume = jnp.sum(ez, axis=1)
    cls = jax.lax.broadcasted_iota(jnp.int32, (8, NUM_CLASSES, L), 1)
    onehot = (cls == conf_t[:, None, :]).astype(f32)
    gathered = jnp.sum(mc3 * onehot, axis=1)
    lse = jnp.log(sume) + cmax
    ce = lse - gathered

    b0, b1 = _channels(bi_conf_ref[0], 2)
    amax = jnp.maximum(b0, b1)
    e0 = jnp.exp(b0 - amax)
    e1 = jnp.exp(b1 - amax)
    score1 = e1 / (e0 + e1)

    v = jnp.where(pos, f32(0.0), ce)
    v = jnp.where(jnp.logical_and(conf_t <= 0, score1 < POS_PRIOR_THRESHOLD),
                  f32(0.0), v)

    k = jnp.minimum(NEG_POS_RATIO * num_pos, P - 1)
    skey = _sortable_key(v)

    def bit_body(i, tbits):
        cand_bits = tbits | (jnp.int32(1) << (31 - i))
        cand = cand_bits ^ jnp.int32(-2147483648)
        cnt = jnp.sum((skey >= cand).astype(jnp.int32))
        return jnp.where(cnt >= k, cand_bits, tbits)

    tbits = jax.lax.fori_loop(0, 32, bit_body, jnp.int32(0))
    tkey = tbits ^ jnp.int32(-2147483648)
    c_gt = jnp.sum((skey > tkey).astype(jnp.int32))
    eq = skey == tkey
    eqi = eq.astype(jnp.int32)
    r = k - c_gt

    def idx_body(i, x):
        cand = x | (jnp.int32(1) << (15 - i))
        cnt = jnp.sum(jnp.where(idx < cand, eqi, 0))
        return jnp.where(cnt < r, cand, x)

    xcut = jax.lax.fori_loop(0, 16, idx_body, jnp.int32(0))
    neg = jnp.logical_or(skey > tkey,
                         jnp.logical_and(eq, idx <= xcut))
    neg = jnp.logical_and(neg, k > 0)

    selm = jnp.logical_or(pos, neg).astype(f32)
    loss_c = jnp.sum(ce * selm)

    ll_ref[0, 0] += loss_l
    lc_ref[0, 0] += loss_c
    np_ref[0, 0] += num_pos.astype(f32)


@jax.jit
def _odm_loss_impl(bi_loc_pred, bi_conf_pred, multi_loc_pred, multi_conf_pred,
                   priors, targets):
    B = bi_loc_pred.shape[0]
    bi_loc_r = jax.lax.stop_gradient(bi_loc_pred).reshape(B, L, 32)
    bi_conf_r = jax.lax.stop_gradient(bi_conf_pred).reshape(B, L, 16)
    multi_loc_r = multi_loc_pred.reshape(B, L, 32)
    multi_conf_r = multi_conf_pred.reshape(B, L, 8 * NUM_CLASSES)
    priors_p = jnp.transpose(
        jax.lax.stop_gradient(priors).reshape(L, 8, 4), (2, 1, 0))
    targets_d = jax.lax.stop_gradient(targets)

    ll, lc, npos = pl.pallas_call(
        _odm_kernel,
        grid=(B,),
        in_specs=[
            pl.BlockSpec((1, T, 5), lambda b: (b, 0, 0),
                         memory_space=pltpu.SMEM),
            pl.BlockSpec((1, L, 32), lambda b: (b, 0, 0)),
            pl.BlockSpec((1, L, 16), lambda b: (b, 0, 0)),
            pl.BlockSpec((1, L, 32), lambda b: (b, 0, 0)),
            pl.BlockSpec((1, L, 8 * NUM_CLASSES), lambda b: (b, 0, 0)),
            pl.BlockSpec((4, 8, L), lambda b: (0, 0, 0)),
        ],
        out_specs=[
            pl.BlockSpec((1, 1), lambda b: (0, 0), memory_space=pltpu.SMEM),
            pl.BlockSpec((1, 1), lambda b: (0, 0), memory_space=pltpu.SMEM),
            pl.BlockSpec((1, 1), lambda b: (0, 0), memory_space=pltpu.SMEM),
        ],
        out_shape=[
            jax.ShapeDtypeStruct((1, 1), jnp.float32),
            jax.ShapeDtypeStruct((1, 1), jnp.float32),
            jax.ShapeDtypeStruct((1, 1), jnp.float32),
        ],
    )(targets_d, bi_loc_r, bi_conf_r, multi_loc_r, multi_conf_r, priors_p)

    total = npos[0, 0]
    return ll[0, 0] / total, lc[0, 0] / total


def kernel(bi_loc_pred, bi_conf_pred, multi_loc_pred, multi_conf_pred,
           priors, targets):
    return _odm_loss_impl(bi_loc_pred, bi_conf_pred, multi_loc_pred,
                          multi_conf_pred, priors, targets)


# speedup vs baseline: 2.0209x; 2.0209x over previous
import jax
import jax.numpy as jnp
from jax.experimental import pallas as pl
from jax.experimental.pallas import tpu as pltpu

NUM_CLASSES = 21
OVERLAP_THRESH = 0.5
NEG_POS_RATIO = 3
ARM_VARIANCE = (0.1, 0.2)
VARIANCE = (0.1, 0.2)
POS_PRIOR_THRESHOLD = 0.01
T = 8
P = 16320
L = P // 8


def _sortable_key(v):
    b = jax.lax.bitcast_convert_type(v, jnp.int32)
    return jnp.where(b >= 0, b, b ^ jnp.int32(0x7FFFFFFF))


def _pack(row):
    return jnp.concatenate(
        [row[:, i * L:(i + 1) * L] for i in range(8)], axis=0)


def _odm_kernel(tgt_ref, bi_loc_ref, bi_conf_ref, multi_loc_ref,
                multi_conf_ref, priors_ref, ll_ref, lc_ref, np_ref):
    f32 = jnp.float32

    @pl.when(pl.program_id(0) == 0)
    def _init():
        ll_ref[0, 0] = f32(0.0)
        lc_ref[0, 0] = f32(0.0)
        np_ref[0, 0] = f32(0.0)

    al = bi_loc_ref[0]
    ml = multi_loc_ref[0]
    bc = bi_conf_ref[0]
    mc = multi_conf_ref[:, 0, 0, :]
    pr = priors_ref[...]

    cx = pr[0:1] + al[0:1] * (ARM_VARIANCE[0] * pr[2:3])
    cy = pr[1:2] + al[1:2] * (ARM_VARIANCE[0] * pr[3:4])
    w = pr[2:3] * jnp.exp(al[2:3] * ARM_VARIANCE[1])
    h = pr[3:4] * jnp.exp(al[3:4] * ARM_VARIANCE[1])
    x0 = cx - w / 2.0
    y0 = cy - h / 2.0
    x1 = cx + w / 2.0
    y1 = cy + h / 2.0
    area_p = (x1 - x0) * (y1 - y0)

    sub8 = jax.lax.broadcasted_iota(jnp.int32, (T, P), 0)
    lane8 = jax.lax.broadcasted_iota(jnp.int32, (T, P), 1)
    zc = jnp.zeros((T, 1), f32)
    tx0c, ty0c, tx1c, ty1c, tlbc = zc, zc, zc, zc, zc
    s1 = sub8[:, 0:1]
    for t in range(T):
        tx0c = jnp.where(s1 == t, tgt_ref[0, t, 0], tx0c)
        ty0c = jnp.where(s1 == t, tgt_ref[0, t, 1], ty0c)
        tx1c = jnp.where(s1 == t, tgt_ref[0, t, 2], tx1c)
        ty1c = jnp.where(s1 == t, tgt_ref[0, t, 3], ty1c)
        tlbc = jnp.where(s1 == t, tgt_ref[0, t, 4], tlbc)

    iw = jnp.maximum(jnp.minimum(tx1c, x1) - jnp.maximum(tx0c, x0), 0.0)
    ih = jnp.maximum(jnp.minimum(ty1c, y1) - jnp.maximum(ty0c, y0), 0.0)
    inter = iw * ih
    area_t = (tx1c - tx0c) * (ty1c - ty0c)
    iou8 = inter / (area_t + area_p - inter)

    bt_over = jnp.max(iou8, axis=0, keepdims=True)
    bt_idx = jnp.min(jnp.where(iou8 == bt_over, sub8, jnp.int32(T)),
                     axis=0, keepdims=True)
    rowmax = jnp.max(iou8, axis=1, keepdims=True)
    bpi = jnp.min(jnp.where(iou8 == rowmax, lane8, jnp.int32(P)),
                  axis=1, keepdims=True)
    hit2 = lane8 == bpi
    anyhit = jnp.max(jnp.where(hit2, 1, 0), axis=0, keepdims=True) > 0
    lastt = jnp.max(jnp.where(hit2, sub8, -1), axis=0, keepdims=True)
    bt_over = jnp.where(anyhit, f32(2.0), bt_over)
    bt_idx = jnp.where(anyhit, lastt, bt_idx)

    oh = bt_idx == sub8
    m0 = jnp.sum(jnp.where(oh, tx0c, 0.0), axis=0, keepdims=True)
    m1 = jnp.sum(jnp.where(oh, ty0c, 0.0), axis=0, keepdims=True)
    m2 = jnp.sum(jnp.where(oh, tx1c, 0.0), axis=0, keepdims=True)
    m3 = jnp.sum(jnp.where(oh, ty1c, 0.0), axis=0, keepdims=True)
    lbl = jnp.sum(jnp.where(oh, tlbc, 0.0), axis=0, keepdims=True)

    conf_t = lbl.astype(jnp.int32) + 1
    conf_t = jnp.where(bt_over < OVERLAP_THRESH, 0, conf_t)
    pos = conf_t > 0
    posf = pos.astype(f32)
    num_pos = jnp.sum(conf_t > 0, dtype=jnp.int32)

    g0 = ((m0 + m2) / 2.0 - cx) / (VARIANCE[0] * w)
    g1 = ((m1 + m3) / 2.0 - cy) / (VARIANCE[0] * h)
    g2 = jnp.log((m2 - m0) / w) / VARIANCE[1]
    g3 = jnp.log((m3 - m1) / h) / VARIANCE[1]

    loss_l = f32(0.0)
    for d, g in ((ml[0:1], g0), (ml[1:2], g1), (ml[2:3], g2), (ml[3:4], g3)):
        df = d - g
        ad = jnp.abs(df)
        sl1 = jnp.where(ad < 1.0, 0.5 * df * df, ad - 0.5)
        loss_l = loss_l + jnp.sum(sl1 * posf)

    cmax = jnp.max(mc, axis=0, keepdims=True)
    ez = jnp.exp(mc - cmax)
    sume = jnp.sum(ez, axis=0, keepdims=True)
    cls = jax.lax.broadcasted_iota(jnp.int32, (NUM_CLASSES, P), 0)
    onehot = cls == conf_t
    gathered = jnp.sum(jnp.where(onehot, mc, 0.0), axis=0, keepdims=True)
    lse = jnp.log(sume) + cmax
    ce = lse - gathered

    amax = jnp.maximum(bc[0:1], bc[1:2])
    e0 = jnp.exp(bc[0:1] - amax)
    e1 = jnp.exp(bc[1:2] - amax)
    score1 = e1 / (e0 + e1)

    v = jnp.where(pos, f32(0.0), ce)
    v = jnp.where(jnp.logical_and(conf_t <= 0, score1 < POS_PRIOR_THRESHOLD),
                  f32(0.0), v)

    k = jnp.minimum(NEG_POS_RATIO * num_pos, P - 1)
    skey = _pack(_sortable_key(v))
    ce_p = _pack(ce)
    pos_p = _pack(posf) > 0.5
    idx = (jax.lax.broadcasted_iota(jnp.int32, (8, L), 0) * L
           + jax.lax.broadcasted_iota(jnp.int32, (8, L), 1))

    def bit_body(i, tbits):
        cand_bits = tbits | (jnp.int32(1) << (31 - i))
        cand = cand_bits ^ jnp.int32(-2147483648)
        cnt = jnp.sum((skey >= cand).astype(jnp.int32))
        return jnp.where(cnt >= k, cand_bits, tbits)

    tbits = jax.lax.fori_loop(0, 32, bit_body, jnp.int32(0))
    tkey = tbits ^ jnp.int32(-2147483648)
    c_gt = jnp.sum((skey > tkey).astype(jnp.int32))
    eq = skey == tkey
    eqi = eq.astype(jnp.int32)
    r = k - c_gt

    def idx_body(i, x):
        cand = x | (jnp.int32(1) << (15 - i))
        cnt = jnp.sum(jnp.where(idx < cand, eqi, 0))
        return jnp.where(cnt < r, cand, x)

    xcut = jax.lax.fori_loop(0, 16, idx_body, jnp.int32(0))
    neg = jnp.logical_or(skey > tkey,
                         jnp.logical_and(eq, idx <= xcut))
    neg = jnp.logical_and(neg, k > 0)

    selm = jnp.logical_or(pos_p, neg).astype(f32)
    loss_c = jnp.sum(ce_p * selm)

    ll_ref[0, 0] += loss_l
    lc_ref[0, 0] += loss_c
    np_ref[0, 0] += num_pos.astype(f32)


@jax.jit
def _odm_loss_impl(bi_loc_pred, bi_conf_pred, multi_loc_pred, multi_conf_pred,
                   priors, targets):
    B = bi_loc_pred.shape[0]
    bl2 = jnp.transpose(jax.lax.stop_gradient(bi_loc_pred), (0, 2, 1))
    bc2 = jnp.transpose(jax.lax.stop_gradient(bi_conf_pred), (0, 2, 1))
    ml2 = jnp.transpose(multi_loc_pred, (0, 2, 1))
    mc2 = jnp.transpose(multi_conf_pred, (2, 0, 1)).reshape(
        NUM_CLASSES, B, 1, P)
    pr2 = jnp.transpose(jax.lax.stop_gradient(priors), (1, 0))
    targets_d = jax.lax.stop_gradient(targets)

    ll, lc, npos = pl.pallas_call(
        _odm_kernel,
        grid=(B,),
        in_specs=[
            pl.BlockSpec((1, T, 5), lambda b: (b, 0, 0),
                         memory_space=pltpu.SMEM),
            pl.BlockSpec((1, 4, P), lambda b: (b, 0, 0)),
            pl.BlockSpec((1, 2, P), lambda b: (b, 0, 0)),
            pl.BlockSpec((1, 4, P), lambda b: (b, 0, 0)),
            pl.BlockSpec((NUM_CLASSES, 1, 1, P), lambda b: (0, b, 0, 0)),
            pl.BlockSpec((4, P), lambda b: (0, 0)),
        ],
        out_specs=[
            pl.BlockSpec((1, 1), lambda b: (0, 0), memory_space=pltpu.SMEM),
            pl.BlockSpec((1, 1), lambda b: (0, 0), memory_space=pltpu.SMEM),
            pl.BlockSpec((1, 1), lambda b: (0, 0), memory_space=pltpu.SMEM),
        ],
        out_shape=[
            jax.ShapeDtypeStruct((1, 1), jnp.float32),
            jax.ShapeDtypeStruct((1, 1), jnp.float32),
            jax.ShapeDtypeStruct((1, 1), jnp.float32),
        ],
    )(targets_d, bl2, bc2, ml2, mc2, pr2)

    total = npos[0, 0]
    return ll[0, 0] / total, lc[0, 0] / total


def kernel(bi_loc_pred, bi_conf_pred, multi_loc_pred, multi_conf_pred,
           priors, targets):
    return _odm_loss_impl(bi_loc_pred, bi_conf_pred, multi_loc_pred,
                          multi_conf_pred, priors, targets)


# speedup vs baseline: 2.6817x; 1.3270x over previous
import jax
import jax.numpy as jnp
from jax.experimental import pallas as pl
from jax.experimental.pallas import tpu as pltpu

NUM_CLASSES = 21
OVERLAP_THRESH = 0.5
NEG_POS_RATIO = 3
ARM_VARIANCE = (0.1, 0.2)
VARIANCE = (0.1, 0.2)
POS_PRIOR_THRESHOLD = 0.01
T = 8
P = 16320
L = P // 8


def _sortable_key(v):
    b = jax.lax.bitcast_convert_type(v, jnp.int32)
    return jnp.where(b >= 0, b, b ^ jnp.int32(0x7FFFFFFF))


def _pack(row):
    return jnp.concatenate(
        [row[:, i * L:(i + 1) * L] for i in range(8)], axis=0)


def _odm_kernel(tgt_ref, bi_loc_ref, bi_conf_ref, multi_loc_ref,
                multi_conf_ref, priors_ref, ll_ref, lc_ref, np_ref):
    f32 = jnp.float32

    @pl.when(pl.program_id(0) == 0)
    def _init():
        ll_ref[0, 0] = f32(0.0)
        lc_ref[0, 0] = f32(0.0)
        np_ref[0, 0] = f32(0.0)

    al = bi_loc_ref[0]
    ml = multi_loc_ref[0]
    bc = bi_conf_ref[0]
    mc = multi_conf_ref[:, 0, 0, :]
    pr = priors_ref[...]

    cx = pr[0:1] + al[0:1] * (ARM_VARIANCE[0] * pr[2:3])
    cy = pr[1:2] + al[1:2] * (ARM_VARIANCE[0] * pr[3:4])
    w = pr[2:3] * jnp.exp(al[2:3] * ARM_VARIANCE[1])
    h = pr[3:4] * jnp.exp(al[3:4] * ARM_VARIANCE[1])
    x0 = cx - w / 2.0
    y0 = cy - h / 2.0
    x1 = cx + w / 2.0
    y1 = cy + h / 2.0
    area_p = (x1 - x0) * (y1 - y0)

    sub8 = jax.lax.broadcasted_iota(jnp.int32, (T, P), 0)
    lane8 = jax.lax.broadcasted_iota(jnp.int32, (T, P), 1)
    zc = jnp.zeros((T, 1), f32)
    tx0c, ty0c, tx1c, ty1c, tlbc = zc, zc, zc, zc, zc
    s1 = sub8[:, 0:1]
    for t in range(T):
        tx0c = jnp.where(s1 == t, tgt_ref[0, t, 0], tx0c)
        ty0c = jnp.where(s1 == t, tgt_ref[0, t, 1], ty0c)
        tx1c = jnp.where(s1 == t, tgt_ref[0, t, 2], tx1c)
        ty1c = jnp.where(s1 == t, tgt_ref[0, t, 3], ty1c)
        tlbc = jnp.where(s1 == t, tgt_ref[0, t, 4], tlbc)

    iw = jnp.maximum(jnp.minimum(tx1c, x1) - jnp.maximum(tx0c, x0), 0.0)
    ih = jnp.maximum(jnp.minimum(ty1c, y1) - jnp.maximum(ty0c, y0), 0.0)
    inter = iw * ih
    area_t = (tx1c - tx0c) * (ty1c - ty0c)
    iou8 = inter / (area_t + area_p - inter)

    bt_over = jnp.max(iou8, axis=0, keepdims=True)
    bt_idx = jnp.min(jnp.where(iou8 == bt_over, sub8, jnp.int32(T)),
                     axis=0, keepdims=True)
    rowmax = jnp.max(iou8, axis=1, keepdims=True)
    bpi = jnp.min(jnp.where(iou8 == rowmax, lane8, jnp.int32(P)),
                  axis=1, keepdims=True)
    hit2 = lane8 == bpi
    anyhit = jnp.max(jnp.where(hit2, 1, 0), axis=0, keepdims=True) > 0
    lastt = jnp.max(jnp.where(hit2, sub8, -1), axis=0, keepdims=True)
    bt_over = jnp.where(anyhit, f32(2.0), bt_over)
    bt_idx = jnp.where(anyhit, lastt, bt_idx)

    ohf = (bt_idx == sub8).astype(f32)
    tm = jnp.concatenate([tx0c, ty0c, tx1c, ty1c, tlbc], axis=1).T
    mm = jnp.dot(tm, ohf, preferred_element_type=f32)
    m0, m1, m2, m3, lbl = (mm[i:i + 1] for i in range(5))

    conf_t = lbl.astype(jnp.int32) + 1
    conf_t = jnp.where(bt_over < OVERLAP_THRESH, 0, conf_t)
    pos = conf_t > 0
    posf = pos.astype(f32)
    num_pos = jnp.sum(conf_t > 0, dtype=jnp.int32)

    g0 = ((m0 + m2) / 2.0 - cx) / (VARIANCE[0] * w)
    g1 = ((m1 + m3) / 2.0 - cy) / (VARIANCE[0] * h)
    g2 = jnp.log((m2 - m0) / w) / VARIANCE[1]
    g3 = jnp.log((m3 - m1) / h) / VARIANCE[1]

    loss_l = f32(0.0)
    for d, g in ((ml[0:1], g0), (ml[1:2], g1), (ml[2:3], g2), (ml[3:4], g3)):
        df = d - g
        ad = jnp.abs(df)
        sl1 = jnp.where(ad < 1.0, 0.5 * df * df, ad - 0.5)
        loss_l = loss_l + jnp.sum(sl1 * posf)

    cmax = jnp.max(mc, axis=0, keepdims=True)
    ez = jnp.exp(mc - cmax)
    cls = jax.lax.broadcasted_iota(jnp.int32, (NUM_CLASSES, P), 0)
    ghsel = jnp.where(cls == conf_t, mc, 0.0)
    ones21 = jnp.ones((1, NUM_CLASSES), f32)
    sume = jnp.dot(ones21, ez, preferred_element_type=f32)
    gathered = jnp.dot(ones21, ghsel, preferred_element_type=f32)
    lse = jnp.log(sume) + cmax
    ce = lse - gathered

    amax = jnp.maximum(bc[0:1], bc[1:2])
    e0 = jnp.exp(bc[0:1] - amax)
    e1 = jnp.exp(bc[1:2] - amax)
    score1 = e1 / (e0 + e1)

    v = jnp.where(pos, f32(0.0), ce)
    v = jnp.where(jnp.logical_and(conf_t <= 0, score1 < POS_PRIOR_THRESHOLD),
                  f32(0.0), v)

    k = jnp.minimum(NEG_POS_RATIO * num_pos, P - 1)
    skey = _pack(_sortable_key(v))
    ce_p = _pack(ce)
    pos_p = _pack(posf) > 0.5
    idx = (jax.lax.broadcasted_iota(jnp.int32, (8, L), 0) * L
           + jax.lax.broadcasted_iota(jnp.int32, (8, L), 1))

    tbits = jnp.int32(0)
    for i in range(32):
        cand_bits = tbits | (jnp.int32(1) << (31 - i))
        cand = cand_bits ^ jnp.int32(-2147483648)
        cnt = jnp.sum((skey >= cand).astype(jnp.int32))
        tbits = jnp.where(cnt >= k, cand_bits, tbits)
    tkey = tbits ^ jnp.int32(-2147483648)
    c_gt = jnp.sum((skey > tkey).astype(jnp.int32))
    eq = skey == tkey
    eqi = eq.astype(jnp.int32)
    ceq = jnp.sum(eqi)
    r = k - c_gt

    def idx_cond(st):
        return jnp.logical_and(st[0] < 16, r < ceq)

    def idx_body(st):
        i, x = st
        cand = x | (jnp.int32(1) << (15 - i))
        cnt = jnp.sum(jnp.where(idx < cand, eqi, 0))
        return i + 1, jnp.where(cnt < r, cand, x)

    _, xs = jax.lax.while_loop(idx_cond, idx_body,
                               (jnp.int32(0), jnp.int32(0)))
    xcut = jnp.where(r < ceq, xs, jnp.int32(P))
    neg = jnp.logical_or(skey > tkey,
                         jnp.logical_and(eq, idx <= xcut))
    neg = jnp.logical_and(neg, k > 0)

    selm = jnp.logical_or(pos_p, neg).astype(f32)
    loss_c = jnp.sum(ce_p * selm)

    ll_ref[0, 0] += loss_l
    lc_ref[0, 0] += loss_c
    np_ref[0, 0] += num_pos.astype(f32)


@jax.jit
def _odm_loss_impl(bi_loc_pred, bi_conf_pred, multi_loc_pred, multi_conf_pred,
                   priors, targets):
    B = bi_loc_pred.shape[0]
    bl2 = jnp.transpose(jax.lax.stop_gradient(bi_loc_pred), (0, 2, 1))
    bc2 = jnp.transpose(jax.lax.stop_gradient(bi_conf_pred), (0, 2, 1))
    ml2 = jnp.transpose(multi_loc_pred, (0, 2, 1))
    mc2 = jnp.transpose(multi_conf_pred, (2, 0, 1)).reshape(
        NUM_CLASSES, B, 1, P)
    pr2 = jnp.transpose(jax.lax.stop_gradient(priors), (1, 0))
    targets_d = jax.lax.stop_gradient(targets)

    ll, lc, npos = pl.pallas_call(
        _odm_kernel,
        grid=(B,),
        in_specs=[
            pl.BlockSpec((1, T, 5), lambda b: (b, 0, 0),
                         memory_space=pltpu.SMEM),
            pl.BlockSpec((1, 4, P), lambda b: (b, 0, 0)),
            pl.BlockSpec((1, 2, P), lambda b: (b, 0, 0)),
            pl.BlockSpec((1, 4, P), lambda b: (b, 0, 0)),
            pl.BlockSpec((NUM_CLASSES, 1, 1, P), lambda b: (0, b, 0, 0)),
            pl.BlockSpec((4, P), lambda b: (0, 0)),
        ],
        out_specs=[
            pl.BlockSpec((1, 1), lambda b: (0, 0), memory_space=pltpu.SMEM),
            pl.BlockSpec((1, 1), lambda b: (0, 0), memory_space=pltpu.SMEM),
            pl.BlockSpec((1, 1), lambda b: (0, 0), memory_space=pltpu.SMEM),
        ],
        out_shape=[
            jax.ShapeDtypeStruct((1, 1), jnp.float32),
            jax.ShapeDtypeStruct((1, 1), jnp.float32),
            jax.ShapeDtypeStruct((1, 1), jnp.float32),
        ],
    )(targets_d, bl2, bc2, ml2, mc2, pr2)

    total = npos[0, 0]
    return ll[0, 0] / total, lc[0, 0] / total


def kernel(bi_loc_pred, bi_conf_pred, multi_loc_pred, multi_conf_pred,
           priors, targets):
    return _odm_loss_impl(bi_loc_pred, bi_conf_pred, multi_loc_pred,
                          multi_conf_pred, priors, targets)


# speedup vs baseline: 4.2606x; 1.5887x over previous
import jax
import jax.numpy as jnp
from jax.experimental import pallas as pl
from jax.experimental.pallas import tpu as pltpu

NUM_CLASSES = 21
OVERLAP_THRESH = 0.5
NEG_POS_RATIO = 3
ARM_VARIANCE = (0.1, 0.2)
VARIANCE = (0.1, 0.2)
POS_PRIOR_THRESHOLD = 0.01
T = 8
P = 16320
B = 32


def _phase1(tgt_ref, bi_loc_ref, bi_conf_ref, multi_loc_ref, multi_conf_ref,
            priors_ref, ce_ref, v_ref, pos_ref, np_ref, ll_ref):
    f32 = jnp.float32
    b = pl.program_id(0)

    @pl.when(b == 0)
    def _init():
        ll_ref[0, 0] = f32(0.0)

    al = bi_loc_ref[0]
    ml = multi_loc_ref[0]
    bc = bi_conf_ref[0]
    mc = multi_conf_ref[:, 0, 0, :]
    pr = priors_ref[...]

    cx = pr[0:1] + al[0:1] * (ARM_VARIANCE[0] * pr[2:3])
    cy = pr[1:2] + al[1:2] * (ARM_VARIANCE[0] * pr[3:4])
    w = pr[2:3] * jnp.exp(al[2:3] * ARM_VARIANCE[1])
    h = pr[3:4] * jnp.exp(al[3:4] * ARM_VARIANCE[1])
    x0 = cx - w / 2.0
    y0 = cy - h / 2.0
    x1 = cx + w / 2.0
    y1 = cy + h / 2.0
    area_p = (x1 - x0) * (y1 - y0)

    sub8 = jax.lax.broadcasted_iota(jnp.int32, (T, P), 0)
    lane8 = jax.lax.broadcasted_iota(jnp.int32, (T, P), 1)
    zc = jnp.zeros((T, 1), f32)
    tx0c, ty0c, tx1c, ty1c, tlbc = zc, zc, zc, zc, zc
    s1 = sub8[:, 0:1]
    for t in range(T):
        tx0c = jnp.where(s1 == t, tgt_ref[0, t, 0], tx0c)
        ty0c = jnp.where(s1 == t, tgt_ref[0, t, 1], ty0c)
        tx1c = jnp.where(s1 == t, tgt_ref[0, t, 2], tx1c)
        ty1c = jnp.where(s1 == t, tgt_ref[0, t, 3], ty1c)
        tlbc = jnp.where(s1 == t, tgt_ref[0, t, 4], tlbc)

    iw = jnp.maximum(jnp.minimum(tx1c, x1) - jnp.maximum(tx0c, x0), 0.0)
    ih = jnp.maximum(jnp.minimum(ty1c, y1) - jnp.maximum(ty0c, y0), 0.0)
    inter = iw * ih
    area_t = (tx1c - tx0c) * (ty1c - ty0c)
    iou8 = inter / (area_t + area_p - inter)

    bt_over = jnp.max(iou8, axis=0, keepdims=True)
    bt_idx = jnp.min(jnp.where(iou8 == bt_over, sub8, jnp.int32(T)),
                     axis=0, keepdims=True)
    rowmax = jnp.max(iou8, axis=1, keepdims=True)
    bpi = jnp.min(jnp.where(iou8 == rowmax, lane8, jnp.int32(P)),
                  axis=1, keepdims=True)
    hit2 = lane8 == bpi
    anyhit = jnp.max(jnp.where(hit2, 1, 0), axis=0, keepdims=True) > 0
    lastt = jnp.max(jnp.where(hit2, sub8, -1), axis=0, keepdims=True)
    bt_over = jnp.where(anyhit, f32(2.0), bt_over)
    bt_idx = jnp.where(anyhit, lastt, bt_idx)

    ohf = (bt_idx == sub8).astype(f32)
    tm = jnp.concatenate([tx0c, ty0c, tx1c, ty1c, tlbc], axis=1).T
    mm = jnp.dot(tm, ohf, preferred_element_type=f32)
    m0, m1, m2, m3, lbl = (mm[i:i + 1] for i in range(5))

    conf_t = lbl.astype(jnp.int32) + 1
    conf_t = jnp.where(bt_over < OVERLAP_THRESH, 0, conf_t)
    pos = conf_t > 0
    posf = pos.astype(f32)
    num_pos = jnp.sum(conf_t > 0, dtype=jnp.int32)

    g0 = ((m0 + m2) / 2.0 - cx) / (VARIANCE[0] * w)
    g1 = ((m1 + m3) / 2.0 - cy) / (VARIANCE[0] * h)
    g2 = jnp.log((m2 - m0) / w) / VARIANCE[1]
    g3 = jnp.log((m3 - m1) / h) / VARIANCE[1]

    loss_l = f32(0.0)
    for d, g in ((ml[0:1], g0), (ml[1:2], g1), (ml[2:3], g2), (ml[3:4], g3)):
        df = d - g
        ad = jnp.abs(df)
        sl1 = jnp.where(ad < 1.0, 0.5 * df * df, ad - 0.5)
        loss_l = loss_l + jnp.sum(sl1 * posf)

    cmax = jnp.max(mc, axis=0, keepdims=True)
    ez = jnp.exp(mc - cmax)
    cls = jax.lax.broadcasted_iota(jnp.int32, (NUM_CLASSES, P), 0)
    ghsel = jnp.where(cls == conf_t, mc, 0.0)
    ones21 = jnp.ones((1, NUM_CLASSES), f32)
    sume = jnp.dot(ones21, ez, preferred_element_type=f32)
    gathered = jnp.dot(ones21, ghsel, preferred_element_type=f32)
    lse = jnp.log(sume) + cmax
    ce = lse - gathered

    amax = jnp.maximum(bc[0:1], bc[1:2])
    e0 = jnp.exp(bc[0:1] - amax)
    e1 = jnp.exp(bc[1:2] - amax)
    score1 = e1 / (e0 + e1)

    v = jnp.where(pos, f32(0.0), ce)
    v = jnp.where(jnp.logical_and(conf_t <= 0, score1 < POS_PRIOR_THRESHOLD),
                  f32(0.0), v)

    rb = jax.lax.rem(b, 8)
    ce_ref[pl.ds(rb, 1), :] = ce
    v_ref[pl.ds(rb, 1), :] = v
    pos_ref[pl.ds(rb, 1), :] = posf
    np_ref[0, 0, 0] = num_pos
    ll_ref[0, 0] += loss_l


def _phase2(ce_ref, v_ref, pos_ref, np_ref, lc_ref, nt_ref):
    f32 = jnp.float32
    ce = ce_ref[...]
    v = v_ref[...]
    posf = pos_ref[...]

    col = jax.lax.broadcasted_iota(jnp.int32, (B, 1), 0)
    npc = jnp.zeros((B, 1), jnp.int32)
    for b in range(B):
        npc = jnp.where(col == b, np_ref[b, 0, 0], npc)
    k = jnp.minimum(NEG_POS_RATIO * npc, P - 1)

    bb = jax.lax.bitcast_convert_type(v, jnp.int32)
    skey = jnp.where(bb >= 0, bb, bb ^ jnp.int32(0x7FFFFFFF))
    idx = jax.lax.broadcasted_iota(jnp.int32, (B, P), 1)

    tbits = jnp.zeros((B, 1), jnp.int32)
    for i in range(32):
        cand_bits = tbits | (jnp.int32(1) << (31 - i))
        cand = cand_bits ^ jnp.int32(-2147483648)
        cnt = jnp.sum((skey >= cand).astype(jnp.int32), axis=1, keepdims=True)
        tbits = jnp.where(cnt >= k, cand_bits, tbits)
    tkey = tbits ^ jnp.int32(-2147483648)
    gt = skey > tkey
    eq = skey == tkey
    eqi = eq.astype(jnp.int32)
    c_gt = jnp.sum(gt.astype(jnp.int32), axis=1, keepdims=True)
    r = k - c_gt
    xcut = jnp.zeros((B, 1), jnp.int32)
    for i in range(16):
        cand = xcut | (jnp.int32(1) << (15 - i))
        cnt = jnp.sum(jnp.where(idx < cand, eqi, 0), axis=1, keepdims=True)
        xcut = jnp.where(cnt < r, cand, xcut)

    neg = jnp.logical_or(gt, jnp.logical_and(eq, idx <= xcut))
    neg = jnp.logical_and(neg, k > 0)
    selm = jnp.logical_or(posf > 0.5, neg).astype(f32)
    lc_ref[0, 0] = jnp.sum(ce * selm)
    nt_ref[0, 0] = jnp.sum(posf)


@jax.jit
def _odm_loss_impl(bi_loc_pred, bi_conf_pred, multi_loc_pred, multi_conf_pred,
                   priors, targets):
    bl2 = jnp.transpose(jax.lax.stop_gradient(bi_loc_pred), (0, 2, 1))
    bc2 = jnp.transpose(jax.lax.stop_gradient(bi_conf_pred), (0, 2, 1))
    ml2 = jnp.transpose(multi_loc_pred, (0, 2, 1))
    mc2 = jnp.transpose(multi_conf_pred, (2, 0, 1)).reshape(
        NUM_CLASSES, B, 1, P)
    pr2 = jnp.transpose(jax.lax.stop_gradient(priors), (1, 0))
    targets_d = jax.lax.stop_gradient(targets)

    ce_a, v_a, pos_a, np_a, ll = pl.pallas_call(
        _phase1,
        grid=(B,),
        in_specs=[
            pl.BlockSpec((1, T, 5), lambda b: (b, 0, 0),
                         memory_space=pltpu.SMEM),
            pl.BlockSpec((1, 4, P), lambda b: (b, 0, 0)),
            pl.BlockSpec((1, 2, P), lambda b: (b, 0, 0)),
            pl.BlockSpec((1, 4, P), lambda b: (b, 0, 0)),
            pl.BlockSpec((NUM_CLASSES, 1, 1, P), lambda b: (0, b, 0, 0)),
            pl.BlockSpec((4, P), lambda b: (0, 0)),
        ],
        out_specs=[
            pl.BlockSpec((8, P), lambda b: (b // 8, 0)),
            pl.BlockSpec((8, P), lambda b: (b // 8, 0)),
            pl.BlockSpec((8, P), lambda b: (b // 8, 0)),
            pl.BlockSpec((1, 1, 1), lambda b: (b, 0, 0), memory_space=pltpu.SMEM),
            pl.BlockSpec((1, 1), lambda b: (0, 0), memory_space=pltpu.SMEM),
        ],
        out_shape=[
            jax.ShapeDtypeStruct((B, P), jnp.float32),
            jax.ShapeDtypeStruct((B, P), jnp.float32),
            jax.ShapeDtypeStruct((B, P), jnp.float32),
            jax.ShapeDtypeStruct((B, 1, 1), jnp.int32),
            jax.ShapeDtypeStruct((1, 1), jnp.float32),
        ],
    )(targets_d, bl2, bc2, ml2, mc2, pr2)

    lc, nt = pl.pallas_call(
        _phase2,
        grid=(1,),
        in_specs=[
            pl.BlockSpec((B, P), lambda i: (0, 0)),
            pl.BlockSpec((B, P), lambda i: (0, 0)),
            pl.BlockSpec((B, P), lambda i: (0, 0)),
            pl.BlockSpec((B, 1, 1), lambda i: (0, 0, 0), memory_space=pltpu.SMEM),
        ],
        out_specs=[
            pl.BlockSpec((1, 1), lambda i: (0, 0), memory_space=pltpu.SMEM),
            pl.BlockSpec((1, 1), lambda i: (0, 0), memory_space=pltpu.SMEM),
        ],
        out_shape=[
            jax.ShapeDtypeStruct((1, 1), jnp.float32),
            jax.ShapeDtypeStruct((1, 1), jnp.float32),
        ],
    )(ce_a, v_a, pos_a, np_a)

    total = nt[0, 0]
    return ll[0, 0] / total, lc[0, 0] / total


def kernel(bi_loc_pred, bi_conf_pred, multi_loc_pred, multi_conf_pred,
           priors, targets):
    return _odm_loss_impl(bi_loc_pred, bi_conf_pred, multi_loc_pred,
                          multi_conf_pred, priors, targets)
